# baseline (speedup 1.0000x reference)
import functools

import jax
import jax.numpy as jnp
from jax import lax
from jax.experimental import pallas as pl
from jax.experimental.pallas import tpu as pltpu
from jax.experimental.pallas import tpu_sc as plsc

HIDDEN = 768
SEQ = 2048
BATCH = 4
EPS = 1e-12

NC = 2
NS = 16
NW = NC * NS
SEQ_PER_W = SEQ // NW
NVEC = HIDDEN // 16


def _rsqrt(x):
    i = plsc.bitcast(x, jnp.int32)
    i = jnp.int32(0x5F3759DF) - lax.shift_right_arithmetic(i, jnp.int32(1))
    y = plsc.bitcast(i, jnp.float32)
    for _ in range(3):
        y = y * (1.5 - 0.5 * x * y * y)
    return y


_GATHER_DNUMS = lax.GatherDimensionNumbers(
    offset_dims=(), collapsed_slice_dims=(0,), start_index_map=(0,))


def _lane_sum(x):
    lanes = lax.iota(jnp.int32, 16)
    for k in (1, 2, 4, 8):
        idx = lax.bitwise_xor(lanes, jnp.int32(k))
        x = x + lax.gather(x, idx[:, None], _GATHER_DNUMS, (1,),
                           mode=lax.GatherScatterMode.PROMISE_IN_BOUNDS)
    return x


def _make_kernel():
    mesh = plsc.VectorSubcoreMesh(core_axis_name="c", subcore_axis_name="s")

    @functools.partial(
        pl.kernel,
        mesh=mesh,
        out_type=jax.ShapeDtypeStruct((BATCH * SEQ, HIDDEN), jnp.float32),
        compiler_params=pltpu.CompilerParams(needs_layout_passes=False),
        scratch_types=[
            pltpu.VMEM((SEQ_PER_W,), jnp.int32),
            pltpu.VMEM((SEQ_PER_W, HIDDEN), jnp.float32),
            pltpu.VMEM((SEQ_PER_W, HIDDEN), jnp.float32),
            pltpu.VMEM((HIDDEN,), jnp.float32),
            pltpu.VMEM((HIDDEN,), jnp.float32),
            pltpu.VMEM((HIDDEN,), jnp.float32),
            pltpu.SemaphoreType.DMA,
        ],
    )
    def emb_kernel(ids_hbm, wemb_hbm, pos_hbm, tt_hbm, w_hbm, b_hbm,
                   out_hbm, idx_v, pos_v, rows_v, tt_v, w_v, b_v, sem):
        wid = lax.axis_index("s") * NC + lax.axis_index("c")
        seq0 = wid * SEQ_PER_W

        pltpu.sync_copy(pos_hbm.at[pl.ds(seq0, SEQ_PER_W)], pos_v)
        pltpu.sync_copy(tt_hbm.at[0], tt_v)
        pltpu.sync_copy(w_hbm, w_v)
        pltpu.sync_copy(b_hbm, b_v)

        def fold_tt(r, _):
            for j in range(NVEC):
                sl = pl.ds(j * 16, 16)
                pos_v[r, sl] = pos_v[r, sl] + tt_v[sl]
            return 0

        lax.fori_loop(0, SEQ_PER_W, fold_tt, 0)

        inv_h = jnp.float32(1.0 / HIDDEN)

        def process_batch(b):
            pltpu.sync_copy(ids_hbm.at[pl.ds(b * SEQ + seq0, SEQ_PER_W)],
                            idx_v)
            pltpu.async_copy(wemb_hbm.at[idx_v], rows_v, sem).wait()

            def row_body(r, _):
                s = jnp.zeros((16,), jnp.float32)
                ss = jnp.zeros((16,), jnp.float32)
                for j in range(NVEC):
                    sl = pl.ds(j * 16, 16)
                    x = rows_v[r, sl] + pos_v[r, sl]
                    rows_v[r, sl] = x
                    s = s + x
                    ss = ss + x * x
                mean = _lane_sum(s) * inv_h
                var = _lane_sum(ss) * inv_h - mean * mean
                rinv = _rsqrt(var + jnp.float32(EPS))
                c0 = -mean * rinv
                for j in range(NVEC):
                    sl = pl.ds(j * 16, 16)
                    t = rows_v[r, sl] * rinv + c0
                    rows_v[r, sl] = t * w_v[sl] + b_v[sl]
                return 0

            lax.fori_loop(0, SEQ_PER_W, row_body, 0)
            pltpu.sync_copy(rows_v,
                            out_hbm.at[pl.ds(b * SEQ + seq0, SEQ_PER_W)])

        for b in range(BATCH):
            process_batch(b)

    return emb_kernel


_EMB_KERNEL = _make_kernel()


def kernel(input_ids, word_embeddings, position_embeddings,
           token_type_embeddings, ln_weight, ln_bias):
    ids_flat = input_ids.reshape(-1)
    out = _EMB_KERNEL(ids_flat, word_embeddings, position_embeddings,
                      token_type_embeddings, ln_weight, ln_bias)
    return out.reshape(BATCH, SEQ, HIDDEN)
